# Initial kernel scaffold; baseline (speedup 1.0000x reference)
import jax
import jax.numpy as jnp
from jax import lax
from jax.experimental import pallas as pl
from jax.experimental.pallas import tpu as pltpu
from jax.experimental.pallas import tpu_sc as plsc

N = 10000
E = 320000
D = 128
G = 64
NC = 2
NS = 16
NW = NC * NS
NPAD = 10016
ROWS_PER_TILE = NPAD // NS

CHUNK = 128
INNER = 4
CPW = 80
EPAD = NW * CPW * CHUNK
NBLK = 10
BLK = N // NBLK


def _sc_agg_body(h_hbm, src_hbm, dst_hbm, zeros_hbm, out_hbm,
                 src_v, dst_v, rows_v, sem):
    c = lax.axis_index("c")
    s = lax.axis_index("s")
    wid = c * NS + s

    def _with_agg(agg_sh):
        pltpu.sync_copy(zeros_hbm, agg_sh.at[pl.ds(s * ROWS_PER_TILE, ROWS_PER_TILE)])
        pltpu.sync_copy(src_hbm.at[wid], src_v)
        pltpu.sync_copy(dst_hbm.at[wid], dst_v)
        plsc.subcore_barrier()

        def round_body(r, carry):
            descs = []
            for j in range(INNER):
                cidx = r * INNER + j
                descs.append(
                    pltpu.async_copy(h_hbm.at[src_v.at[cidx]], rows_v.at[j], sem))
            for d in descs:
                d.wait()
            for j in range(INNER):
                cidx = r * INNER + j
                pltpu.sync_copy(rows_v.at[j], agg_sh.at[dst_v.at[cidx]], add=True)
            return carry

        lax.fori_loop(0, CPW // INNER, round_body, 0)

        plsc.subcore_barrier()
        pltpu.sync_copy(agg_sh.at[pl.ds(s * ROWS_PER_TILE, ROWS_PER_TILE)],
                        out_hbm.at[c, pl.ds(s * ROWS_PER_TILE, ROWS_PER_TILE)])

    pl.run_scoped(_with_agg, pltpu.VMEM_SHARED((NPAD, D), jnp.float32))


def _sc_agg(h, src3, dst3, zeros):
    k = pl.kernel(
        _sc_agg_body,
        out_type=jax.ShapeDtypeStruct((NC, NPAD, D), jnp.float32),
        mesh=plsc.VectorSubcoreMesh(core_axis_name="c", subcore_axis_name="s"),
        scratch_types=[
            pltpu.VMEM((CPW, CHUNK), jnp.int32),
            pltpu.VMEM((CPW, CHUNK), jnp.int32),
            pltpu.VMEM((INNER, CHUNK, D), jnp.float32),
            pltpu.SemaphoreType.DMA,
        ],
    )
    return k(h, src3, dst3, zeros)


def _tc_layer_body(h_ref, a_ref, w_ref, b_ref, o_ref):
    z = h_ref[...] + a_ref[0] + a_ref[1]
    acc = jnp.dot(z, w_ref[...], preferred_element_type=jnp.float32)
    o_ref[...] = jnp.maximum(acc + b_ref[...], 0.0)


def _tc_layer(h, agg, W, b2d):
    return pl.pallas_call(
        _tc_layer_body,
        grid=(NBLK,),
        in_specs=[
            pl.BlockSpec((BLK, D), lambda i: (i, 0)),
            pl.BlockSpec((NC, BLK, D), lambda i: (0, i, 0)),
            pl.BlockSpec((D, D), lambda i: (0, 0)),
            pl.BlockSpec((1, D), lambda i: (0, 0)),
        ],
        out_specs=pl.BlockSpec((BLK, D), lambda i: (i, 0)),
        out_shape=jax.ShapeDtypeStruct((N, D), jnp.float32),
    )(h, agg, W, b2d)


def _tc_final_body(h_ref, a_ref, w3_ref, b3_ref, bat_ref, wo_ref, bo_ref,
                   o_ref, pool_ref, cnt_ref):
    i = pl.program_id(0)

    @pl.when(i == 0)
    def _init():
        pool_ref[...] = jnp.zeros_like(pool_ref)
        cnt_ref[...] = jnp.zeros_like(cnt_ref)

    z = h_ref[...] + a_ref[0] + a_ref[1]
    h3 = jnp.maximum(
        jnp.dot(z, w3_ref[...], preferred_element_type=jnp.float32) + b3_ref[...],
        0.0)
    bat = bat_ref[0]
    gids = lax.broadcasted_iota(jnp.int32, (G, BLK), 0)
    onehot = (gids == jnp.broadcast_to(bat, (G, BLK))).astype(jnp.float32)
    pool_ref[...] += jnp.dot(onehot, h3, preferred_element_type=jnp.float32)
    cnt_ref[...] += jnp.broadcast_to(jnp.sum(onehot, axis=1)[:, None], (G, D))

    @pl.when(i == pl.num_programs(0) - 1)
    def _finish():
        pooled = pool_ref[...] / jnp.maximum(cnt_ref[...], 1.0)
        o_ref[...] = (jnp.dot(pooled, wo_ref[...], preferred_element_type=jnp.float32)
                      + bo_ref[...])


def _tc_final(h, agg, W3, b3_2d, bat3, Wout, bout2d):
    return pl.pallas_call(
        _tc_final_body,
        grid=(NBLK,),
        in_specs=[
            pl.BlockSpec((BLK, D), lambda i: (i, 0)),
            pl.BlockSpec((NC, BLK, D), lambda i: (0, i, 0)),
            pl.BlockSpec((D, D), lambda i: (0, 0)),
            pl.BlockSpec((1, D), lambda i: (0, 0)),
            pl.BlockSpec((1, 1, BLK), lambda i: (i, 0, 0)),
            pl.BlockSpec((D, D), lambda i: (0, 0)),
            pl.BlockSpec((1, D), lambda i: (0, 0)),
        ],
        out_specs=pl.BlockSpec((G, D), lambda i: (0, 0)),
        out_shape=jax.ShapeDtypeStruct((G, D), jnp.float32),
        scratch_shapes=[
            pltpu.VMEM((G, D), jnp.float32),
            pltpu.VMEM((G, D), jnp.float32),
        ],
    )(h, agg, W3, b3_2d, bat3, Wout, bout2d)


def kernel(x, edge_index, batch, W1, b1, W2, b2, W3, b3, Wout, bout):
    src = edge_index[0]
    dst = edge_index[1]
    pad = EPAD - E
    src3 = jnp.concatenate([src, jnp.zeros((pad,), jnp.int32)]).reshape(NW, CPW, CHUNK)
    dst3 = jnp.concatenate([dst, jnp.full((pad,), N, jnp.int32)]).reshape(NW, CPW, CHUNK)
    zeros = jnp.zeros((ROWS_PER_TILE, D), jnp.float32)
    bat3 = batch.reshape(NBLK, 1, BLK)
    b1r, b2r, b3r, boutr = (v.reshape(1, D) for v in (b1, b2, b3, bout))

    agg = _sc_agg(x, src3, dst3, zeros)
    h1 = _tc_layer(x, agg, W1, b1r)
    agg = _sc_agg(h1, src3, dst3, zeros)
    h2 = _tc_layer(h1, agg, W2, b2r)
    agg = _sc_agg(h2, src3, dst3, zeros)
    return _tc_final(h2, agg, W3, b3r, bat3, Wout, boutr)


# speedup vs baseline: 2.6892x; 2.6892x over previous
import jax
import jax.numpy as jnp
from jax import lax
from jax.experimental import pallas as pl
from jax.experimental.pallas import tpu as pltpu
from jax.experimental.pallas import tpu_sc as plsc

N = 10000
E = 320000
D = 128
G = 64
NS = 16
NPAD = 10112
ROWS_PER_TILE = NPAD // NS

CHUNK = 128
INNER = 2
NWORK = NS
CPW = 160
NROUND = CPW // INNER
EPAD = NWORK * CPW * CHUNK
NBLK = 10
BLK = N // NBLK

ZROWS = 79


def _sc_agg_body(h_hbm, src_hbm, dst_hbm, out_hbm, agg_sh, sem, isem):
    s = lax.axis_index("s")
    wid = s

    def _inner(rows_v, srci, dsti, zbuf):
        zero16 = jnp.zeros((16,), jnp.float32)

        def zfill(k, carry):
            for m in range(8):
                zbuf[k, pl.ds(m * 16, 16)] = zero16
            return carry

        lax.fori_loop(0, ZROWS, zfill, 0)
        for p in range(8):
            pltpu.sync_copy(
                zbuf, agg_sh.at[pl.ds(s * ROWS_PER_TILE + p * ZROWS, ZROWS)])
        plsc.subcore_barrier()

        pltpu.async_copy(src_hbm.at[wid, pl.ds(0, INNER)], srci.at[0], isem).wait()
        pltpu.async_copy(dst_hbm.at[wid, pl.ds(0, INNER)], dsti.at[0], isem).wait()

        def round_body(r, carry):
            pr = r % 2
            nxt = (r + 1) % 2
            rpf = jnp.minimum(r + 1, NROUND - 1)
            pf = [
                pltpu.async_copy(
                    src_hbm.at[wid, pl.ds(rpf * INNER, INNER)], srci.at[nxt], isem),
                pltpu.async_copy(
                    dst_hbm.at[wid, pl.ds(rpf * INNER, INNER)], dsti.at[nxt], isem),
            ]

            descs = []
            for j in range(INNER):
                descs.append(
                    pltpu.async_copy(h_hbm.at[srci.at[pr, j]], rows_v.at[j], sem))
            for d in descs:
                d.wait()
            for j in range(INNER):
                pltpu.sync_copy(rows_v.at[j], agg_sh.at[dsti.at[pr, j]], add=True)
            for d in pf:
                d.wait()
            return carry

        lax.fori_loop(0, NROUND, round_body, 0)

        plsc.subcore_barrier()
        for p, nrows in ((0, CHUNK), (1, CHUNK), (2, CHUNK), (3, CHUNK),
                         (4, ROWS_PER_TILE - 4 * CHUNK)):
            off = s * ROWS_PER_TILE + p * CHUNK
            stage = rows_v.at[p % INNER].at[pl.ds(0, nrows)]
            pltpu.sync_copy(agg_sh.at[pl.ds(off, nrows)], stage)
            pltpu.sync_copy(stage, out_hbm.at[pl.ds(off, nrows)])

    pl.run_scoped(_inner,
                  pltpu.VMEM((INNER, CHUNK, D), jnp.float32),
                  pltpu.VMEM((2, INNER, CHUNK), jnp.int32),
                  pltpu.VMEM((2, INNER, CHUNK), jnp.int32),
                  pltpu.VMEM((ZROWS, D), jnp.float32))


def _sc_agg(h, src3, dst3):
    k = pl.kernel(
        _sc_agg_body,
        out_type=jax.ShapeDtypeStruct((NPAD, D), jnp.float32),
        mesh=plsc.VectorSubcoreMesh(core_axis_name="c", subcore_axis_name="s",
                                    num_cores=1),
        scratch_types=[
            pltpu.VMEM_SHARED((NPAD, D), jnp.float32),
            pltpu.SemaphoreType.DMA,
            pltpu.SemaphoreType.DMA,
        ],
    )
    return k(h, src3, dst3)


def _tc_layer_body(h_ref, a_ref, w_ref, b_ref, o_ref):
    z = h_ref[...] + a_ref[...]
    acc = jnp.dot(z, w_ref[...], preferred_element_type=jnp.float32)
    o_ref[...] = jnp.maximum(acc + b_ref[...], 0.0)


def _tc_layer(h, agg, W, b2d):
    return pl.pallas_call(
        _tc_layer_body,
        grid=(NBLK,),
        in_specs=[
            pl.BlockSpec((BLK, D), lambda i: (i, 0)),
            pl.BlockSpec((BLK, D), lambda i: (i, 0)),
            pl.BlockSpec((D, D), lambda i: (0, 0)),
            pl.BlockSpec((1, D), lambda i: (0, 0)),
        ],
        out_specs=pl.BlockSpec((BLK, D), lambda i: (i, 0)),
        out_shape=jax.ShapeDtypeStruct((N, D), jnp.float32),
    )(h, agg, W, b2d)


def _tc_final_body(h_ref, a_ref, w3_ref, b3_ref, bat_ref, wo_ref, bo_ref,
                   o_ref, pool_ref, cnt_ref):
    i = pl.program_id(0)

    @pl.when(i == 0)
    def _init():
        pool_ref[...] = jnp.zeros_like(pool_ref)
        cnt_ref[...] = jnp.zeros_like(cnt_ref)

    z = h_ref[...] + a_ref[...]
    h3 = jnp.maximum(
        jnp.dot(z, w3_ref[...], preferred_element_type=jnp.float32) + b3_ref[...],
        0.0)
    bat = bat_ref[0]
    gids = lax.broadcasted_iota(jnp.int32, (G, BLK), 0)
    onehot = (gids == jnp.broadcast_to(bat, (G, BLK))).astype(jnp.float32)
    pool_ref[...] += jnp.dot(onehot, h3, preferred_element_type=jnp.float32)
    cnt_ref[...] += jnp.broadcast_to(jnp.sum(onehot, axis=1)[:, None], (G, D))

    @pl.when(i == pl.num_programs(0) - 1)
    def _finish():
        pooled = pool_ref[...] / jnp.maximum(cnt_ref[...], 1.0)
        o_ref[...] = (jnp.dot(pooled, wo_ref[...], preferred_element_type=jnp.float32)
                      + bo_ref[...])


def _tc_final(h, agg, W3, b3_2d, bat3, Wout, bout2d):
    return pl.pallas_call(
        _tc_final_body,
        grid=(NBLK,),
        in_specs=[
            pl.BlockSpec((BLK, D), lambda i: (i, 0)),
            pl.BlockSpec((BLK, D), lambda i: (i, 0)),
            pl.BlockSpec((D, D), lambda i: (0, 0)),
            pl.BlockSpec((1, D), lambda i: (0, 0)),
            pl.BlockSpec((1, 1, BLK), lambda i: (i, 0, 0)),
            pl.BlockSpec((D, D), lambda i: (0, 0)),
            pl.BlockSpec((1, D), lambda i: (0, 0)),
        ],
        out_specs=pl.BlockSpec((G, D), lambda i: (0, 0)),
        out_shape=jax.ShapeDtypeStruct((G, D), jnp.float32),
        scratch_shapes=[
            pltpu.VMEM((G, D), jnp.float32),
            pltpu.VMEM((G, D), jnp.float32),
        ],
    )(h, agg, W3, b3_2d, bat3, Wout, bout2d)


def kernel(x, edge_index, batch, W1, b1, W2, b2, W3, b3, Wout, bout):
    src = edge_index[0]
    dst = edge_index[1]
    pad = EPAD - E
    src3 = jnp.concatenate([src, jnp.zeros((pad,), jnp.int32)]).reshape(NWORK, CPW, CHUNK)
    dst3 = jnp.concatenate([dst, jnp.full((pad,), N, jnp.int32)]).reshape(NWORK, CPW, CHUNK)
    bat3 = batch.reshape(NBLK, 1, BLK)
    b1r, b2r, b3r, boutr = (v.reshape(1, D) for v in (b1, b2, b3, bout))

    agg = _sc_agg(x, src3, dst3)
    h1 = _tc_layer(x, agg, W1, b1r)
    agg = _sc_agg(h1, src3, dst3)
    h2 = _tc_layer(h1, agg, W2, b2r)
    agg = _sc_agg(h2, src3, dst3)
    return _tc_final(h2, agg, W3, b3r, bat3, Wout, boutr)


# speedup vs baseline: 4.0326x; 1.4995x over previous
import jax
import jax.numpy as jnp
from jax import lax
from jax.experimental import pallas as pl
from jax.experimental.pallas import tpu as pltpu
from jax.experimental.pallas import tpu_sc as plsc

N = 10000
E = 320000
D = 128
F = 64
G = 64
NC = 2
NS = 16
NPAD = 10112
ROWS_PER_TILE = NPAD // NS

CHUNK = 128
INNER = 4
NWORK = NS
CPW = 160
NROUND = CPW // INNER
EPAD = NWORK * CPW * CHUNK
NBLK = 10
BLK = N // NBLK

ZROWS = 79


def _sc_agg_body(h_hbm, src_hbm, dst_hbm, out_hbm, agg_sh, sem, isem):
    c = lax.axis_index("c")
    s = lax.axis_index("s")
    wid = s
    hc = h_hbm.at[c]
    outc = out_hbm.at[c]

    def _inner(rows_v, srci, dsti, zbuf):
        zero16 = jnp.zeros((16,), jnp.float32)

        def zfill(k, carry):
            for m in range(4):
                zbuf[k, pl.ds(m * 16, 16)] = zero16
            return carry

        lax.fori_loop(0, ZROWS, zfill, 0)
        for p in range(8):
            pltpu.sync_copy(
                zbuf, agg_sh.at[pl.ds(s * ROWS_PER_TILE + p * ZROWS, ZROWS)])
        plsc.subcore_barrier()

        pltpu.async_copy(src_hbm.at[wid, pl.ds(0, INNER)], srci.at[0], isem).wait()
        pltpu.async_copy(dst_hbm.at[wid, pl.ds(0, INNER)], dsti.at[0], isem).wait()

        def round_body(r, carry):
            pr = r % 2
            nxt = (r + 1) % 2
            rpf = jnp.minimum(r + 1, NROUND - 1)
            pf = [
                pltpu.async_copy(
                    src_hbm.at[wid, pl.ds(rpf * INNER, INNER)], srci.at[nxt], isem),
                pltpu.async_copy(
                    dst_hbm.at[wid, pl.ds(rpf * INNER, INNER)], dsti.at[nxt], isem),
            ]

            descs = []
            for j in range(INNER):
                descs.append(
                    pltpu.async_copy(hc.at[srci.at[pr, j]], rows_v.at[j], sem))
            for d in descs:
                d.wait()
            for j in range(INNER):
                pltpu.sync_copy(rows_v.at[j], agg_sh.at[dsti.at[pr, j]], add=True)
            for d in pf:
                d.wait()
            return carry

        lax.fori_loop(0, NROUND, round_body, 0)

        plsc.subcore_barrier()
        for p, nrows in ((0, CHUNK), (1, CHUNK), (2, CHUNK), (3, CHUNK),
                         (4, ROWS_PER_TILE - 4 * CHUNK)):
            off = s * ROWS_PER_TILE + p * CHUNK
            stage = rows_v.at[p % INNER].at[pl.ds(0, nrows)]
            pltpu.sync_copy(agg_sh.at[pl.ds(off, nrows)], stage)
            pltpu.sync_copy(stage, outc.at[pl.ds(off, nrows)])

    pl.run_scoped(_inner,
                  pltpu.VMEM((INNER, CHUNK, F), jnp.float32),
                  pltpu.VMEM((2, INNER, CHUNK), jnp.int32),
                  pltpu.VMEM((2, INNER, CHUNK), jnp.int32),
                  pltpu.VMEM((ZROWS, F), jnp.float32))


def _sc_agg(h2, src3, dst3):
    k = pl.kernel(
        _sc_agg_body,
        out_type=jax.ShapeDtypeStruct((NC, NPAD, F), jnp.float32),
        mesh=plsc.VectorSubcoreMesh(core_axis_name="c", subcore_axis_name="s"),
        compiler_params=pltpu.CompilerParams(use_tc_tiling_on_sc=False),
        scratch_types=[
            pltpu.VMEM_SHARED((NPAD, F), jnp.float32),
            pltpu.SemaphoreType.DMA,
            pltpu.SemaphoreType.DMA,
        ],
    )
    return k(h2, src3, dst3)


def _tc_layer_body(h_ref, a_ref, w_ref, b_ref, o_ref):
    z = (jnp.concatenate([h_ref[0], h_ref[1]], axis=1)
         + jnp.concatenate([a_ref[0], a_ref[1]], axis=1))
    acc = jnp.dot(z, w_ref[...], preferred_element_type=jnp.float32)
    h = jnp.maximum(acc + b_ref[...], 0.0)
    o_ref[0] = h[:, :F]
    o_ref[1] = h[:, F:]


def _tc_layer(h2, agg, W, b2d):
    return pl.pallas_call(
        _tc_layer_body,
        grid=(NBLK,),
        in_specs=[
            pl.BlockSpec((NC, BLK, F), lambda i: (0, i, 0)),
            pl.BlockSpec((NC, BLK, F), lambda i: (0, i, 0)),
            pl.BlockSpec((D, D), lambda i: (0, 0)),
            pl.BlockSpec((1, D), lambda i: (0, 0)),
        ],
        out_specs=pl.BlockSpec((NC, BLK, F), lambda i: (0, i, 0)),
        out_shape=jax.ShapeDtypeStruct((NC, N, F), jnp.float32),
    )(h2, agg, W, b2d)


def _tc_final_body(h_ref, a_ref, w3_ref, b3_ref, bat_ref, wo_ref, bo_ref,
                   o_ref, pool_ref, cnt_ref):
    i = pl.program_id(0)

    @pl.when(i == 0)
    def _init():
        pool_ref[...] = jnp.zeros_like(pool_ref)
        cnt_ref[...] = jnp.zeros_like(cnt_ref)

    z = (jnp.concatenate([h_ref[0], h_ref[1]], axis=1)
         + jnp.concatenate([a_ref[0], a_ref[1]], axis=1))
    h3 = jnp.maximum(
        jnp.dot(z, w3_ref[...], preferred_element_type=jnp.float32) + b3_ref[...],
        0.0)
    bat = bat_ref[0]
    gids = lax.broadcasted_iota(jnp.int32, (G, BLK), 0)
    onehot = (gids == jnp.broadcast_to(bat, (G, BLK))).astype(jnp.float32)
    pool_ref[...] += jnp.dot(onehot, h3, preferred_element_type=jnp.float32)
    cnt_ref[...] += jnp.broadcast_to(jnp.sum(onehot, axis=1)[:, None], (G, D))

    @pl.when(i == pl.num_programs(0) - 1)
    def _finish():
        pooled = pool_ref[...] / jnp.maximum(cnt_ref[...], 1.0)
        o_ref[...] = (jnp.dot(pooled, wo_ref[...], preferred_element_type=jnp.float32)
                      + bo_ref[...])


def _tc_final(h2, agg, W3, b3_2d, bat3, Wout, bout2d):
    return pl.pallas_call(
        _tc_final_body,
        grid=(NBLK,),
        in_specs=[
            pl.BlockSpec((NC, BLK, F), lambda i: (0, i, 0)),
            pl.BlockSpec((NC, BLK, F), lambda i: (0, i, 0)),
            pl.BlockSpec((D, D), lambda i: (0, 0)),
            pl.BlockSpec((1, D), lambda i: (0, 0)),
            pl.BlockSpec((1, 1, BLK), lambda i: (i, 0, 0)),
            pl.BlockSpec((D, D), lambda i: (0, 0)),
            pl.BlockSpec((1, D), lambda i: (0, 0)),
        ],
        out_specs=pl.BlockSpec((G, D), lambda i: (0, 0)),
        out_shape=jax.ShapeDtypeStruct((G, D), jnp.float32),
        scratch_shapes=[
            pltpu.VMEM((G, D), jnp.float32),
            pltpu.VMEM((G, D), jnp.float32),
        ],
    )(h2, agg, W3, b3_2d, bat3, Wout, bout2d)


def kernel(x, edge_index, batch, W1, b1, W2, b2, W3, b3, Wout, bout):
    src = edge_index[0]
    dst = edge_index[1]
    pad = EPAD - E
    src3 = jnp.concatenate([src, jnp.zeros((pad,), jnp.int32)]).reshape(NWORK, CPW, CHUNK)
    dst3 = jnp.concatenate([dst, jnp.full((pad,), N, jnp.int32)]).reshape(NWORK, CPW, CHUNK)
    bat3 = batch.reshape(NBLK, 1, BLK)
    b1r, b2r, b3r, boutr = (v.reshape(1, D) for v in (b1, b2, b3, bout))
    x2 = jnp.stack([x[:, :F], x[:, F:]])

    agg = _sc_agg(x2, src3, dst3)
    h1 = _tc_layer(x2, agg, W1, b1r)
    agg = _sc_agg(h1, src3, dst3)
    h2 = _tc_layer(h1, agg, W2, b2r)
    agg = _sc_agg(h2, src3, dst3)
    return _tc_final(h2, agg, W3, b3r, bat3, Wout, boutr)


# speedup vs baseline: 4.4687x; 1.1082x over previous
import jax
import jax.numpy as jnp
from jax import lax
from jax.experimental import pallas as pl
from jax.experimental.pallas import tpu as pltpu
from jax.experimental.pallas import tpu_sc as plsc

N = 10000
E = 320000
D = 128
F = 64
G = 64
NC = 2
NS = 16
NPAD = 10112
ROWS_PER_TILE = NPAD // NS

CHUNK = 128
INNER = 4
NWORK = NS
CPW = 160
NROUND = CPW // INNER
EPAD = NWORK * CPW * CHUNK
NBLK = 10
BLK = N // NBLK

ZROWS = 79


def _sc_agg_body(h_hbm, src_hbm, dst_hbm, out_hbm, agg_sh,
                 gsem0, gsem1, ssem0, ssem1, isem_s, isem_d):
    c = lax.axis_index("c")
    s = lax.axis_index("s")
    wid = s
    hc = h_hbm.at[c]
    outc = out_hbm.at[c]
    gsem = (gsem0, gsem1)
    ssem = (ssem0, ssem1)

    def _inner(rows_v, srci, dsti, zbuf):
        zero16 = jnp.zeros((16,), jnp.float32)

        def zfill(k, carry):
            for m in range(4):
                zbuf[k, pl.ds(m * 16, 16)] = zero16
            return carry

        lax.fori_loop(0, ZROWS, zfill, 0)
        for p in range(8):
            pltpu.sync_copy(
                zbuf, agg_sh.at[pl.ds(s * ROWS_PER_TILE + p * ZROWS, ZROWS)])
        plsc.subcore_barrier()

        def fire_gathers(g, st):
            for j in range(INNER):
                pltpu.async_copy(hc.at[srci.at[st, j]], rows_v.at[st, j], gsem[st])

        def fire_scatters(st):
            for j in range(INNER):
                pltpu.async_copy(rows_v.at[st, j], agg_sh.at[dsti.at[st, j]],
                                 ssem[st], add=True)

        def drain_rows(sem, st):
            for j in range(INNER):
                pltpu.make_async_copy(hc.at[pl.ds(0, CHUNK)],
                                      rows_v.at[st, j], sem).wait()

        def drain_idx(sem, buf, st):
            pltpu.make_async_copy(src_hbm.at[wid, pl.ds(0, INNER)],
                                  buf.at[st], sem).wait()

        def fetch_idx(g, buf, hbm, st, sem):
            pltpu.async_copy(hbm.at[wid, pl.ds(g * INNER, INNER)], buf.at[st], sem)

        fetch_idx(0, srci, src_hbm, 0, isem_s)
        drain_idx(isem_s, srci, 0)
        fetch_idx(0, dsti, dst_hbm, 0, isem_d)
        fire_gathers(0, 0)
        fetch_idx(1, srci, src_hbm, 1, isem_s)

        def half_round(r, cur, nxt):
            @pl.when(r >= 1)
            def _():
                drain_rows(ssem[nxt], nxt)
            drain_rows(gsem[cur], cur)
            drain_idx(isem_d, dsti, cur)
            fire_scatters(cur)

            @pl.when(r + 1 < NROUND)
            def _():
                fetch_idx(r + 1, dsti, dst_hbm, nxt, isem_d)
                drain_idx(isem_s, srci, nxt)
                fire_gathers(r + 1, nxt)

            @pl.when(r + 2 < NROUND)
            def _():
                fetch_idx(r + 2, srci, src_hbm, cur, isem_s)

        def round_body(t, carry):
            half_round(2 * t, 0, 1)
            half_round(2 * t + 1, 1, 0)
            return carry

        lax.fori_loop(0, NROUND // 2, round_body, 0)
        drain_rows(ssem[(NROUND - 1) % 2], (NROUND - 1) % 2)

        plsc.subcore_barrier()
        for p, nrows in ((0, CHUNK), (1, CHUNK), (2, CHUNK), (3, CHUNK),
                         (4, ROWS_PER_TILE - 4 * CHUNK)):
            off = s * ROWS_PER_TILE + p * CHUNK
            stage = rows_v.at[0].at[p % INNER].at[pl.ds(0, nrows)]
            pltpu.sync_copy(agg_sh.at[pl.ds(off, nrows)], stage)
            pltpu.sync_copy(stage, outc.at[pl.ds(off, nrows)])

    pl.run_scoped(_inner,
                  pltpu.VMEM((2, INNER, CHUNK, F), jnp.float32),
                  pltpu.VMEM((2, INNER, CHUNK), jnp.int32),
                  pltpu.VMEM((2, INNER, CHUNK), jnp.int32),
                  pltpu.VMEM((ZROWS, F), jnp.float32))


def _sc_agg(h2, src3, dst3):
    k = pl.kernel(
        _sc_agg_body,
        out_type=jax.ShapeDtypeStruct((NC, NPAD, F), jnp.float32),
        mesh=plsc.VectorSubcoreMesh(core_axis_name="c", subcore_axis_name="s"),
        compiler_params=pltpu.CompilerParams(use_tc_tiling_on_sc=False),
        scratch_types=[
            pltpu.VMEM_SHARED((NPAD, F), jnp.float32),
            pltpu.SemaphoreType.DMA,
            pltpu.SemaphoreType.DMA,
            pltpu.SemaphoreType.DMA,
            pltpu.SemaphoreType.DMA,
            pltpu.SemaphoreType.DMA,
            pltpu.SemaphoreType.DMA,
        ],
    )
    return k(h2, src3, dst3)


def _tc_layer_body(h_ref, a_ref, w_ref, b_ref, o_ref):
    z = (jnp.concatenate([h_ref[0], h_ref[1]], axis=1)
         + jnp.concatenate([a_ref[0], a_ref[1]], axis=1))
    acc = jnp.dot(z, w_ref[...], preferred_element_type=jnp.float32)
    h = jnp.maximum(acc + b_ref[...], 0.0)
    o_ref[0] = h[:, :F]
    o_ref[1] = h[:, F:]


def _tc_layer(h2, agg, W, b2d):
    return pl.pallas_call(
        _tc_layer_body,
        grid=(NBLK,),
        in_specs=[
            pl.BlockSpec((NC, BLK, F), lambda i: (0, i, 0)),
            pl.BlockSpec((NC, BLK, F), lambda i: (0, i, 0)),
            pl.BlockSpec((D, D), lambda i: (0, 0)),
            pl.BlockSpec((1, D), lambda i: (0, 0)),
        ],
        out_specs=pl.BlockSpec((NC, BLK, F), lambda i: (0, i, 0)),
        out_shape=jax.ShapeDtypeStruct((NC, N, F), jnp.float32),
    )(h2, agg, W, b2d)


def _tc_final_body(h_ref, a_ref, w3_ref, b3_ref, bat_ref, wo_ref, bo_ref,
                   o_ref, pool_ref, cnt_ref):
    i = pl.program_id(0)

    @pl.when(i == 0)
    def _init():
        pool_ref[...] = jnp.zeros_like(pool_ref)
        cnt_ref[...] = jnp.zeros_like(cnt_ref)

    z = (jnp.concatenate([h_ref[0], h_ref[1]], axis=1)
         + jnp.concatenate([a_ref[0], a_ref[1]], axis=1))
    h3 = jnp.maximum(
        jnp.dot(z, w3_ref[...], preferred_element_type=jnp.float32) + b3_ref[...],
        0.0)
    bat = bat_ref[0]
    gids = lax.broadcasted_iota(jnp.int32, (G, BLK), 0)
    onehot = (gids == jnp.broadcast_to(bat, (G, BLK))).astype(jnp.float32)
    pool_ref[...] += jnp.dot(onehot, h3, preferred_element_type=jnp.float32)
    cnt_ref[...] += jnp.broadcast_to(jnp.sum(onehot, axis=1)[:, None], (G, D))

    @pl.when(i == pl.num_programs(0) - 1)
    def _finish():
        pooled = pool_ref[...] / jnp.maximum(cnt_ref[...], 1.0)
        o_ref[...] = (jnp.dot(pooled, wo_ref[...], preferred_element_type=jnp.float32)
                      + bo_ref[...])


def _tc_final(h2, agg, W3, b3_2d, bat3, Wout, bout2d):
    return pl.pallas_call(
        _tc_final_body,
        grid=(NBLK,),
        in_specs=[
            pl.BlockSpec((NC, BLK, F), lambda i: (0, i, 0)),
            pl.BlockSpec((NC, BLK, F), lambda i: (0, i, 0)),
            pl.BlockSpec((D, D), lambda i: (0, 0)),
            pl.BlockSpec((1, D), lambda i: (0, 0)),
            pl.BlockSpec((1, 1, BLK), lambda i: (i, 0, 0)),
            pl.BlockSpec((D, D), lambda i: (0, 0)),
            pl.BlockSpec((1, D), lambda i: (0, 0)),
        ],
        out_specs=pl.BlockSpec((G, D), lambda i: (0, 0)),
        out_shape=jax.ShapeDtypeStruct((G, D), jnp.float32),
        scratch_shapes=[
            pltpu.VMEM((G, D), jnp.float32),
            pltpu.VMEM((G, D), jnp.float32),
        ],
    )(h2, agg, W3, b3_2d, bat3, Wout, bout2d)


def kernel(x, edge_index, batch, W1, b1, W2, b2, W3, b3, Wout, bout):
    src = edge_index[0]
    dst = edge_index[1]
    pad = EPAD - E
    src3 = jnp.concatenate([src, jnp.zeros((pad,), jnp.int32)]).reshape(NWORK, CPW, CHUNK)
    dst3 = jnp.concatenate([dst, jnp.full((pad,), N, jnp.int32)]).reshape(NWORK, CPW, CHUNK)
    bat3 = batch.reshape(NBLK, 1, BLK)
    b1r, b2r, b3r, boutr = (v.reshape(1, D) for v in (b1, b2, b3, bout))
    x2 = jnp.stack([x[:, :F], x[:, F:]])

    agg = _sc_agg(x2, src3, dst3)
    h1 = _tc_layer(x2, agg, W1, b1r)
    agg = _sc_agg(h1, src3, dst3)
    h2 = _tc_layer(h1, agg, W2, b2r)
    agg = _sc_agg(h2, src3, dst3)
    return _tc_final(h2, agg, W3, b3r, bat3, Wout, boutr)


# speedup vs baseline: 8.8522x; 1.9809x over previous
import jax
import jax.numpy as jnp
from jax import lax
from jax.experimental import pallas as pl
from jax.experimental.pallas import tpu as pltpu
from jax.experimental.pallas import tpu_sc as plsc

N = 10000
E = 320000
D = 128
F = 64
G = 64
NC = 2
NS = 16
NPAD = 10112
ROWS_PER_TILE = NPAD // NS

CHUNK = 128
INNER = 2
NWORK = NS
CPW = 160
NROUND = CPW // INNER
EPAD = NWORK * CPW * CHUNK
NBLK = 10
BLK = N // NBLK

ZROWS = 79


def _sc_agg_body(h_hbm, src_hbm, dst_hbm, out_hbm, agg_sh, h_sh,
                 gsem0, gsem1, ssem0, ssem1, isem_s, isem_d):
    c = lax.axis_index("c")
    s = lax.axis_index("s")
    wid = s
    hc = h_hbm.at[c]
    outc = out_hbm.at[c]
    gsem = (gsem0, gsem1)
    ssem = (ssem0, ssem1)

    def _inner(rows_v, srci, dsti, zbuf):
        zero16 = jnp.zeros((16,), jnp.float32)

        def zfill(k, carry):
            for m in range(4):
                zbuf[k, pl.ds(m * 16, 16)] = zero16
            return carry

        lax.fori_loop(0, ZROWS, zfill, 0)
        for p in range(8):
            pltpu.sync_copy(
                zbuf, agg_sh.at[pl.ds(s * ROWS_PER_TILE + p * ZROWS, ZROWS)])
        for p, nrows in ((0, CHUNK), (1, CHUNK), (2, CHUNK), (3, CHUNK),
                         (4, ROWS_PER_TILE - 4 * CHUNK)):
            off = s * ROWS_PER_TILE + p * CHUNK
            stage = rows_v.at[0].at[p % INNER].at[pl.ds(0, nrows)]
            pltpu.sync_copy(hc.at[pl.ds(off, nrows)], stage)
            pltpu.sync_copy(stage, h_sh.at[pl.ds(off, nrows)])
        plsc.subcore_barrier()

        def fire_gathers(g, st):
            for j in range(INNER):
                pltpu.async_copy(h_sh.at[srci.at[st, j]], rows_v.at[st, j],
                                 gsem[st])

        def fire_scatters(st):
            for j in range(INNER):
                pltpu.async_copy(rows_v.at[st, j], agg_sh.at[dsti.at[st, j]],
                                 ssem[st], add=True)

        def drain_rows(sem, st):
            for j in range(INNER):
                pltpu.make_async_copy(hc.at[pl.ds(0, CHUNK)],
                                      rows_v.at[st, j], sem).wait()

        def drain_idx(sem, buf, st):
            pltpu.make_async_copy(src_hbm.at[wid, pl.ds(0, INNER)],
                                  buf.at[st], sem).wait()

        def fetch_idx(g, buf, hbm, st, sem):
            pltpu.async_copy(hbm.at[wid, pl.ds(g * INNER, INNER)], buf.at[st], sem)

        fetch_idx(0, srci, src_hbm, 0, isem_s)
        drain_idx(isem_s, srci, 0)
        fetch_idx(0, dsti, dst_hbm, 0, isem_d)
        fire_gathers(0, 0)
        fetch_idx(1, srci, src_hbm, 1, isem_s)

        def half_round(r, cur, nxt):
            @pl.when(r >= 1)
            def _():
                drain_rows(ssem[nxt], nxt)
            drain_rows(gsem[cur], cur)
            drain_idx(isem_d, dsti, cur)
            fire_scatters(cur)

            @pl.when(r + 1 < NROUND)
            def _():
                fetch_idx(r + 1, dsti, dst_hbm, nxt, isem_d)
                drain_idx(isem_s, srci, nxt)
                fire_gathers(r + 1, nxt)

            @pl.when(r + 2 < NROUND)
            def _():
                fetch_idx(r + 2, srci, src_hbm, cur, isem_s)

        def round_body(t, carry):
            half_round(2 * t, 0, 1)
            half_round(2 * t + 1, 1, 0)
            return carry

        lax.fori_loop(0, NROUND // 2, round_body, 0)
        drain_rows(ssem[(NROUND - 1) % 2], (NROUND - 1) % 2)

        plsc.subcore_barrier()
        for p, nrows in ((0, CHUNK), (1, CHUNK), (2, CHUNK), (3, CHUNK),
                         (4, ROWS_PER_TILE - 4 * CHUNK)):
            off = s * ROWS_PER_TILE + p * CHUNK
            stage = rows_v.at[0].at[p % INNER].at[pl.ds(0, nrows)]
            pltpu.sync_copy(agg_sh.at[pl.ds(off, nrows)], stage)
            pltpu.sync_copy(stage, outc.at[pl.ds(off, nrows)])

    pl.run_scoped(_inner,
                  pltpu.VMEM((2, INNER, CHUNK, F), jnp.float32),
                  pltpu.VMEM((2, INNER, CHUNK), jnp.int32),
                  pltpu.VMEM((2, INNER, CHUNK), jnp.int32),
                  pltpu.VMEM((ZROWS, F), jnp.float32))


def _sc_agg(h2, src3, dst3):
    k = pl.kernel(
        _sc_agg_body,
        out_type=jax.ShapeDtypeStruct((NC, NPAD, F), jnp.float32),
        mesh=plsc.VectorSubcoreMesh(core_axis_name="c", subcore_axis_name="s"),
        compiler_params=pltpu.CompilerParams(use_tc_tiling_on_sc=False),
        scratch_types=[
            pltpu.VMEM_SHARED((NPAD, F), jnp.float32),
            pltpu.VMEM_SHARED((NPAD, F), jnp.float32),
            pltpu.SemaphoreType.DMA,
            pltpu.SemaphoreType.DMA,
            pltpu.SemaphoreType.DMA,
            pltpu.SemaphoreType.DMA,
            pltpu.SemaphoreType.DMA,
            pltpu.SemaphoreType.DMA,
        ],
    )
    return k(h2, src3, dst3)


def _tc_layer_body(h_ref, a_ref, w_ref, b_ref, o_ref):
    z = (jnp.concatenate([h_ref[0], h_ref[1]], axis=1)
         + jnp.concatenate([a_ref[0], a_ref[1]], axis=1))
    acc = jnp.dot(z, w_ref[...], preferred_element_type=jnp.float32)
    h = jnp.maximum(acc + b_ref[...], 0.0)
    o_ref[0] = h[:, :F]
    o_ref[1] = h[:, F:]


def _tc_layer(h2, agg, W, b2d):
    return pl.pallas_call(
        _tc_layer_body,
        grid=(NBLK,),
        in_specs=[
            pl.BlockSpec((NC, BLK, F), lambda i: (0, i, 0)),
            pl.BlockSpec((NC, BLK, F), lambda i: (0, i, 0)),
            pl.BlockSpec((D, D), lambda i: (0, 0)),
            pl.BlockSpec((1, D), lambda i: (0, 0)),
        ],
        out_specs=pl.BlockSpec((NC, BLK, F), lambda i: (0, i, 0)),
        out_shape=jax.ShapeDtypeStruct((NC, NPAD, F), jnp.float32),
    )(h2, agg, W, b2d)


def _tc_final_body(h_ref, a_ref, w3_ref, b3_ref, bat_ref, wo_ref, bo_ref,
                   o_ref, pool_ref, cnt_ref):
    i = pl.program_id(0)

    @pl.when(i == 0)
    def _init():
        pool_ref[...] = jnp.zeros_like(pool_ref)
        cnt_ref[...] = jnp.zeros_like(cnt_ref)

    z = (jnp.concatenate([h_ref[0], h_ref[1]], axis=1)
         + jnp.concatenate([a_ref[0], a_ref[1]], axis=1))
    h3 = jnp.maximum(
        jnp.dot(z, w3_ref[...], preferred_element_type=jnp.float32) + b3_ref[...],
        0.0)
    bat = bat_ref[0]
    gids = lax.broadcasted_iota(jnp.int32, (G, BLK), 0)
    onehot = (gids == jnp.broadcast_to(bat, (G, BLK))).astype(jnp.float32)
    pool_ref[...] += jnp.dot(onehot, h3, preferred_element_type=jnp.float32)
    cnt_ref[...] += jnp.broadcast_to(jnp.sum(onehot, axis=1)[:, None], (G, D))

    @pl.when(i == pl.num_programs(0) - 1)
    def _finish():
        pooled = pool_ref[...] / jnp.maximum(cnt_ref[...], 1.0)
        o_ref[...] = (jnp.dot(pooled, wo_ref[...], preferred_element_type=jnp.float32)
                      + bo_ref[...])


def _tc_final(h2, agg, W3, b3_2d, bat3, Wout, bout2d):
    return pl.pallas_call(
        _tc_final_body,
        grid=(NBLK,),
        in_specs=[
            pl.BlockSpec((NC, BLK, F), lambda i: (0, i, 0)),
            pl.BlockSpec((NC, BLK, F), lambda i: (0, i, 0)),
            pl.BlockSpec((D, D), lambda i: (0, 0)),
            pl.BlockSpec((1, D), lambda i: (0, 0)),
            pl.BlockSpec((1, 1, BLK), lambda i: (i, 0, 0)),
            pl.BlockSpec((D, D), lambda i: (0, 0)),
            pl.BlockSpec((1, D), lambda i: (0, 0)),
        ],
        out_specs=pl.BlockSpec((G, D), lambda i: (0, 0)),
        out_shape=jax.ShapeDtypeStruct((G, D), jnp.float32),
        scratch_shapes=[
            pltpu.VMEM((G, D), jnp.float32),
            pltpu.VMEM((G, D), jnp.float32),
        ],
    )(h2, agg, W3, b3_2d, bat3, Wout, bout2d)


def kernel(x, edge_index, batch, W1, b1, W2, b2, W3, b3, Wout, bout):
    src = edge_index[0]
    dst = edge_index[1]
    pad = EPAD - E
    src3 = jnp.concatenate([src, jnp.zeros((pad,), jnp.int32)]).reshape(NWORK, CPW, CHUNK)
    dst3 = jnp.concatenate([dst, jnp.full((pad,), N, jnp.int32)]).reshape(NWORK, CPW, CHUNK)
    bat3 = batch.reshape(NBLK, 1, BLK)
    b1r, b2r, b3r, boutr = (v.reshape(1, D) for v in (b1, b2, b3, bout))
    x2 = jnp.zeros((NC, NPAD, F), jnp.float32).at[:, :N].set(
        jnp.stack([x[:, :F], x[:, F:]]))

    agg = _sc_agg(x2, src3, dst3)
    h1 = _tc_layer(x2, agg, W1, b1r)
    agg = _sc_agg(h1, src3, dst3)
    h2 = _tc_layer(h1, agg, W2, b2r)
    agg = _sc_agg(h2, src3, dst3)
    return _tc_final(h2, agg, W3, b3r, bat3, Wout, boutr)


# speedup vs baseline: 9.1625x; 1.0351x over previous
import jax
import jax.numpy as jnp
from jax import lax
from jax.experimental import pallas as pl
from jax.experimental.pallas import tpu as pltpu
from jax.experimental.pallas import tpu_sc as plsc

N = 10000
E = 320000
D = 128
F = 64
G = 64
NC = 2
NS = 16
NPAD = 10112
ROWS_PER_TILE = NPAD // NS

CHUNK = 128
INNER = 2
NWORK = NS
CPW = 160
NROUND = CPW // INNER
EPAD = NWORK * CPW * CHUNK
NBLK = 10
BLK = N // NBLK

ZROWS = 79


def _sc_agg_body(h_hbm, src_hbm, dst_hbm, out_hbm, agg_sh, h_sh,
                 gsem0, gsem1, ssem0, ssem1, isem_s, isem_d):
    c = lax.axis_index("c")
    s = lax.axis_index("s")
    wid = s
    hc = h_hbm.at[c]
    outc = out_hbm.at[c]
    gsem = (gsem0, gsem1)
    ssem = (ssem0, ssem1)

    PIECES = ((0, CHUNK), (1, CHUNK), (2, CHUNK), (3, CHUNK),
              (4, ROWS_PER_TILE - 4 * CHUNK))
    BUFS = ((0, 0), (0, 1), (1, 0), (1, 1))

    def _inner(rows_v, srci, dsti):
        live = {}
        for p, (pp, nrows) in enumerate(PIECES):
            b = BUFS[p % 4]
            for d in live.pop(b, ()):
                d.wait()
            off = s * ROWS_PER_TILE + pp * CHUNK
            stage = rows_v.at[b[0], b[1]].at[pl.ds(0, nrows)]
            pltpu.sync_copy(hc.at[pl.ds(off, nrows)], stage)
            live[b] = [
                pltpu.async_copy(stage, h_sh.at[pl.ds(off, nrows)], gsem0),
                pltpu.async_copy(stage, agg_sh.at[pl.ds(off, nrows)], ssem0),
            ]
        for ds_ in live.values():
            for d in ds_:
                d.wait()
        plsc.subcore_barrier()

        def fire_gathers(g, st):
            for j in range(INNER):
                pltpu.async_copy(h_sh.at[srci.at[st, j]], rows_v.at[st, j],
                                 gsem[st])

        def fire_scatters(st):
            for j in range(INNER):
                pltpu.async_copy(rows_v.at[st, j], agg_sh.at[dsti.at[st, j]],
                                 ssem[st], add=True)

        def drain_rows(sem, st):
            for j in range(INNER):
                pltpu.make_async_copy(hc.at[pl.ds(0, CHUNK)],
                                      rows_v.at[st, j], sem).wait()

        def drain_idx(sem, buf, st):
            pltpu.make_async_copy(src_hbm.at[wid, pl.ds(0, INNER)],
                                  buf.at[st], sem).wait()

        def fetch_idx(g, buf, hbm, st, sem):
            pltpu.async_copy(hbm.at[wid, pl.ds(g * INNER, INNER)], buf.at[st], sem)

        fetch_idx(0, srci, src_hbm, 0, isem_s)
        drain_idx(isem_s, srci, 0)
        fetch_idx(0, dsti, dst_hbm, 0, isem_d)
        fire_gathers(0, 0)
        fetch_idx(1, srci, src_hbm, 1, isem_s)

        def half_round(r, cur, nxt):
            @pl.when(r >= 1)
            def _():
                drain_rows(ssem[nxt], nxt)
            drain_rows(gsem[cur], cur)
            drain_idx(isem_d, dsti, cur)
            fire_scatters(cur)

            @pl.when(r + 1 < NROUND)
            def _():
                fetch_idx(r + 1, dsti, dst_hbm, nxt, isem_d)
                drain_idx(isem_s, srci, nxt)
                fire_gathers(r + 1, nxt)

            @pl.when(r + 2 < NROUND)
            def _():
                fetch_idx(r + 2, srci, src_hbm, cur, isem_s)

        def round_body(t, carry):
            half_round(2 * t, 0, 1)
            half_round(2 * t + 1, 1, 0)
            return carry

        lax.fori_loop(0, NROUND // 2, round_body, 0)
        drain_rows(ssem[(NROUND - 1) % 2], (NROUND - 1) % 2)

        plsc.subcore_barrier()
        live = {}
        for p, (pp, nrows) in enumerate(PIECES):
            b = BUFS[p % 4]
            for d in live.pop(b, ()):
                d.wait()
            off = s * ROWS_PER_TILE + pp * CHUNK
            stage = rows_v.at[b[0], b[1]].at[pl.ds(0, nrows)]
            pltpu.sync_copy(agg_sh.at[pl.ds(off, nrows)], stage)
            live[b] = [pltpu.async_copy(stage, outc.at[pl.ds(off, nrows)], gsem1)]
        for ds_ in live.values():
            for d in ds_:
                d.wait()

    pl.run_scoped(_inner,
                  pltpu.VMEM((2, INNER, CHUNK, F), jnp.float32),
                  pltpu.VMEM((2, INNER, CHUNK), jnp.int32),
                  pltpu.VMEM((2, INNER, CHUNK), jnp.int32))


def _sc_agg(h2, src3, dst3):
    k = pl.kernel(
        _sc_agg_body,
        out_type=jax.ShapeDtypeStruct((NC, NPAD, F), jnp.float32),
        mesh=plsc.VectorSubcoreMesh(core_axis_name="c", subcore_axis_name="s"),
        compiler_params=pltpu.CompilerParams(use_tc_tiling_on_sc=False),
        scratch_types=[
            pltpu.VMEM_SHARED((NPAD, F), jnp.float32),
            pltpu.VMEM_SHARED((NPAD, F), jnp.float32),
            pltpu.SemaphoreType.DMA,
            pltpu.SemaphoreType.DMA,
            pltpu.SemaphoreType.DMA,
            pltpu.SemaphoreType.DMA,
            pltpu.SemaphoreType.DMA,
            pltpu.SemaphoreType.DMA,
        ],
    )
    return k(h2, src3, dst3)


def _tc_layer_body(a_ref, w_ref, b_ref, o_ref):
    z = jnp.concatenate([a_ref[0], a_ref[1]], axis=1)
    acc = jnp.dot(z, w_ref[...], preferred_element_type=jnp.float32)
    h = jnp.maximum(acc + b_ref[...], 0.0)
    o_ref[0] = h[:, :F]
    o_ref[1] = h[:, F:]


def _tc_layer(agg, W, b2d):
    return pl.pallas_call(
        _tc_layer_body,
        grid=(NBLK,),
        in_specs=[
            pl.BlockSpec((NC, BLK, F), lambda i: (0, i, 0)),
            pl.BlockSpec((D, D), lambda i: (0, 0)),
            pl.BlockSpec((1, D), lambda i: (0, 0)),
        ],
        out_specs=pl.BlockSpec((NC, BLK, F), lambda i: (0, i, 0)),
        out_shape=jax.ShapeDtypeStruct((NC, NPAD, F), jnp.float32),
    )(agg, W, b2d)


def _tc_final_body(a_ref, w3_ref, b3_ref, bat_ref, wo_ref, bo_ref,
                   o_ref, pool_ref, cnt_ref):
    i = pl.program_id(0)

    @pl.when(i == 0)
    def _init():
        pool_ref[...] = jnp.zeros_like(pool_ref)
        cnt_ref[...] = jnp.zeros_like(cnt_ref)

    z = jnp.concatenate([a_ref[0], a_ref[1]], axis=1)
    h3 = jnp.maximum(
        jnp.dot(z, w3_ref[...], preferred_element_type=jnp.float32) + b3_ref[...],
        0.0)
    bat = bat_ref[0]
    gids = lax.broadcasted_iota(jnp.int32, (G, BLK), 0)
    onehot = (gids == jnp.broadcast_to(bat, (G, BLK))).astype(jnp.float32)
    pool_ref[...] += jnp.dot(onehot, h3, preferred_element_type=jnp.float32)
    cnt_ref[...] += jnp.broadcast_to(jnp.sum(onehot, axis=1)[:, None], (G, D))

    @pl.when(i == pl.num_programs(0) - 1)
    def _finish():
        pooled = pool_ref[...] / jnp.maximum(cnt_ref[...], 1.0)
        o_ref[...] = (jnp.dot(pooled, wo_ref[...], preferred_element_type=jnp.float32)
                      + bo_ref[...])


def _tc_final(agg, W3, b3_2d, bat3, Wout, bout2d):
    return pl.pallas_call(
        _tc_final_body,
        grid=(NBLK,),
        in_specs=[
            pl.BlockSpec((NC, BLK, F), lambda i: (0, i, 0)),
            pl.BlockSpec((D, D), lambda i: (0, 0)),
            pl.BlockSpec((1, D), lambda i: (0, 0)),
            pl.BlockSpec((1, 1, BLK), lambda i: (i, 0, 0)),
            pl.BlockSpec((D, D), lambda i: (0, 0)),
            pl.BlockSpec((1, D), lambda i: (0, 0)),
        ],
        out_specs=pl.BlockSpec((G, D), lambda i: (0, 0)),
        out_shape=jax.ShapeDtypeStruct((G, D), jnp.float32),
        scratch_shapes=[
            pltpu.VMEM((G, D), jnp.float32),
            pltpu.VMEM((G, D), jnp.float32),
        ],
    )(agg, W3, b3_2d, bat3, Wout, bout2d)


def kernel(x, edge_index, batch, W1, b1, W2, b2, W3, b3, Wout, bout):
    src = edge_index[0]
    dst = edge_index[1]
    pad = EPAD - E
    src3 = jnp.concatenate([src, jnp.zeros((pad,), jnp.int32)]).reshape(NWORK, CPW, CHUNK)
    dst3 = jnp.concatenate([dst, jnp.full((pad,), N, jnp.int32)]).reshape(NWORK, CPW, CHUNK)
    bat3 = batch.reshape(NBLK, 1, BLK)
    b1r, b2r, b3r, boutr = (v.reshape(1, D) for v in (b1, b2, b3, bout))
    x2 = jnp.zeros((NC, NPAD, F), jnp.float32).at[:, :N].set(
        jnp.stack([x[:, :F], x[:, F:]]))

    agg = _sc_agg(x2, src3, dst3)
    h1 = _tc_layer(agg, W1, b1r)
    agg = _sc_agg(h1, src3, dst3)
    h2 = _tc_layer(agg, W2, b2r)
    agg = _sc_agg(h2, src3, dst3)
    return _tc_final(agg, W3, b3r, bat3, Wout, boutr)
